# baseline (speedup 1.0000x reference)
import jax
import jax.numpy as jnp
from jax import lax
from jax.experimental import pallas as pl
from jax.experimental.pallas import tpu as pltpu
from jax.experimental.pallas import tpu_sc as plsc

_T, _K, _V, _E = 128, 32, 8192, 16
_TEMP = 0.07
_NEG = 32

_R = _T * _K
_NB = 512
_NW = 32
_RPW = _R // _NW
_NV = _V // 16


def _perm(v, idx):
    return v.at[idx].get(mode="promise_in_bounds")


def _sc_tau_body(beta_ref, tau_ref, rowbuf, bktbuf, onesbuf, histv,
                 zerosv, taubuf, selres, shared_hist):
    cid = lax.axis_index("c")
    sid = lax.axis_index("s")
    wid = sid * 2 + cid
    base = wid * _RPW
    region = sid * _NB
    lanes = lax.iota(jnp.int32, 16)
    ninf = jnp.float32(-jnp.inf)
    zero_f = jnp.zeros((16,), jnp.float32)

    def vmaxall(v):
        for s in (8, 4, 2, 1):
            v = jnp.maximum(v, _perm(v, lanes ^ s))
        return v

    def vminall(v):
        for s in (8, 4, 2, 1):
            v = jnp.minimum(v, _perm(v, lanes ^ s))
        return v

    def vsumall(v):
        for s in (8, 4, 2, 1):
            v = v + _perm(v, lanes ^ s)
        return v

    ones_f = jnp.ones((16,), jnp.float32)

    def init_ones(i, c):
        for j in range(16):
            onesbuf[pl.ds((i * 16 + j) * 16, 16)] = ones_f
        return c
    lax.fori_loop(0, _NV // 16, init_ones, 0)

    def init_zeros(i, c):
        for j in range(2):
            zerosv[pl.ds((i * 2 + j) * 16, 16)] = zero_f
        return c
    lax.fori_loop(0, _NB // 32, init_zeros, 0)

    def row_body(r, tau_acc):
        pltpu.sync_copy(beta_ref.at[base + r], rowbuf)

        def p1(i, mm):
            mx, mn = mm
            for j in range(16):
                v = rowbuf[pl.ds((i * 16 + j) * 16, 16)]
                mx = jnp.maximum(mx, v)
                mn = jnp.minimum(mn, v)
            return mx, mn
        mxv, mnv = lax.fori_loop(
            0, _NV // 16, p1,
            (jnp.full((16,), -jnp.inf, jnp.float32),
             jnp.full((16,), jnp.inf, jnp.float32)))
        rmax = vmaxall(mxv)
        rmin = vminall(mnv)
        scale = _NB / jnp.maximum(rmax - rmin, 1e-35)

        pltpu.sync_copy(zerosv, shared_hist.at[pl.ds(region, _NB)])

        def p2(i, c):
            for j in range(16):
                v = rowbuf[pl.ds((i * 16 + j) * 16, 16)]
                b = jnp.minimum(((rmax - v) * scale).astype(jnp.int32),
                                _NB - 1)
                bktbuf[pl.ds((i * 16 + j) * 16, 16)] = b + region
            return c
        lax.fori_loop(0, _NV // 16, p2, 0)

        pltpu.sync_copy(onesbuf, shared_hist.at[bktbuf], add=True)
        pltpu.sync_copy(shared_hist.at[pl.ds(region, _NB)], histv)

        def scan_body(c, carry):
            cum, done, c_star, cum_b = carry
            s = vsumall(histv[pl.ds(c * 16, 16)])[0]
            found_now = jnp.logical_and(done == 0, (cum + s) >= _NEG)
            c_star = jnp.where(found_now, c, c_star)
            cum_b = jnp.where(found_now, cum, cum_b)
            done = jnp.where(found_now, jnp.int32(1), done)
            return cum + s, done, c_star, cum_b

        _, _, c_star, cum_b = lax.fori_loop(
            0, _NB // 16, scan_body,
            (jnp.float32(0.0), jnp.int32(0), jnp.int32(0),
             jnp.float32(0.0)))

        tot = histv[pl.ds(c_star * 16, 16)]
        prefix = tot
        for s in (1, 2, 4, 8):
            sh = _perm(prefix, jnp.maximum(lanes - s, 0))
            prefix = prefix + jnp.where(lanes >= s, sh, 0.0)
        cross = (cum_b + prefix) >= _NEG
        b_lane = vminall(jnp.where(cross, lanes, 16))
        below = jnp.where(lanes < b_lane, tot, 0.0)
        count_above = cum_b + vsumall(below)[0]
        m0 = _NEG - count_above
        b_star = c_star * 16 + b_lane + region

        def sel_body(_it, carry):
            m, hi, tau = carry

            def sp(i, mc):
                mxv_, cntv_ = mc
                for j in range(16):
                    v = rowbuf[pl.ds((i * 16 + j) * 16, 16)]
                    b = bktbuf[pl.ds((i * 16 + j) * 16, 16)]
                    valid = jnp.logical_and(b == b_star, v < hi)
                    cand = jnp.where(valid, v, ninf)
                    mx2 = jnp.maximum(mxv_, cand)
                    hit = jnp.logical_and(valid, cand == mx2)
                    cntv_ = jnp.where(cand > mxv_, 0.0, cntv_) + \
                        jnp.where(hit, 1.0, 0.0)
                    mxv_ = mx2
                return mxv_, cntv_

            @pl.when(m > 0.0)
            def _run_pass():
                mxv0, cntv0 = lax.fori_loop(
                    0, _NV // 16, sp,
                    (jnp.full((16,), -jnp.inf, jnp.float32), zero_f))
                selres[pl.ds(0, 16)] = mxv0
                selres[pl.ds(16, 16)] = cntv0

            mxv_ = selres[pl.ds(0, 16)]
            cntv_ = selres[pl.ds(16, 16)]
            mxs = vmaxall(mxv_)
            mx = mxs[0]
            c_tot = vsumall(jnp.where(mxv_ == mxs, cntv_, 0.0))[0]
            tau = jnp.where(jnp.logical_and(m > 0.0, m <= c_tot), mx, tau)
            return m - c_tot, mx, tau

        _, _, tau = lax.fori_loop(
            0, _NEG, sel_body,
            (m0, jnp.float32(jnp.inf), jnp.float32(0.0)))

        tau_acc = jnp.where(lanes == (r % 16), tau, tau_acc)

        @pl.when(r % 16 == 15)
        def _flush():
            taubuf[pl.ds((r // 16) * 16, 16)] = tau_acc

        return tau_acc

    lax.fori_loop(0, _RPW, row_body, zero_f)
    pltpu.sync_copy(taubuf, tau_ref.at[pl.ds(base, _RPW)])


def _tc_body(tau_ref, tw_ref, beta_ref, temb_ref, wemb_ref, out_ref, acc_ref):
    t = pl.program_id(0)
    X = beta_ref[0]
    tw = tw_ref[0]
    tau = tau_ref[0]

    member = jnp.any(X >= tau, axis=0, keepdims=True)
    negm = member & (tw == 0)

    a = temb_ref[0]
    a = a / (jnp.sqrt(jnp.sum(a * a, axis=-1, keepdims=True)) + 1e-12)
    b = wemb_ref[...]
    b = b / (jnp.sqrt(jnp.sum(b * b, axis=-1, keepdims=True)) + 1e-12)
    sim = jax.lax.dot_general(
        a, b, (((1,), (1,)), ((), ())),
        preferred_element_type=jnp.float32) / _TEMP

    sim_m = jnp.where(negm, sim, -1e9)
    m = jnp.max(sim_m, axis=1, keepdims=True)
    lse = jnp.log(jnp.sum(jnp.exp(sim_m - m), axis=1, keepdims=True)) + m
    loss_t = jnp.sum(lse) / _K
    valid = jnp.any(negm)

    @pl.when(t == 0)
    def _init():
        acc_ref[0] = 0.0
        acc_ref[1] = 0.0

    acc_ref[0] += jnp.where(valid, loss_t, 0.0)
    acc_ref[1] += valid.astype(jnp.float32)

    @pl.when(t == _T - 1)
    def _fin():
        cnt = acc_ref[1]
        out_ref[0, 0] = jnp.where(
            cnt > 0.0, acc_ref[0] / jnp.maximum(cnt, 1.0), 0.0)


def kernel(time_wordcount, beta, topic_embeddings, word_embeddings):
    beta2d = beta.reshape(_R, _V)
    mesh = plsc.VectorSubcoreMesh(core_axis_name="c", subcore_axis_name="s")
    tau = pl.kernel(
        _sc_tau_body,
        mesh=mesh,
        out_type=jax.ShapeDtypeStruct((_R,), jnp.float32),
        scratch_types=[
            pltpu.VMEM((_V,), jnp.float32),
            pltpu.VMEM((_V,), jnp.int32),
            pltpu.VMEM((_V,), jnp.float32),
            pltpu.VMEM((_NB,), jnp.float32),
            pltpu.VMEM((_NB,), jnp.float32),
            pltpu.VMEM((_RPW,), jnp.float32),
            pltpu.VMEM((32,), jnp.float32),
            pltpu.VMEM_SHARED((16 * _NB,), jnp.float32),
        ],
    )(beta2d)

    tau3 = tau.reshape(_T, _K, 1)
    tw3 = time_wordcount.reshape(_T, 1, _V)
    out = pl.pallas_call(
        _tc_body,
        grid=(_T,),
        in_specs=[
            pl.BlockSpec((1, _K, 1), lambda t: (t, 0, 0)),
            pl.BlockSpec((1, 1, _V), lambda t: (t, 0, 0)),
            pl.BlockSpec((1, _K, _V), lambda t: (t, 0, 0)),
            pl.BlockSpec((1, _K, _E), lambda t: (t, 0, 0)),
            pl.BlockSpec((_V, _E), lambda t: (0, 0)),
        ],
        out_specs=pl.BlockSpec(memory_space=pltpu.SMEM),
        out_shape=jax.ShapeDtypeStruct((1, 1), jnp.float32),
        scratch_shapes=[pltpu.SMEM((2,), jnp.float32)],
        compiler_params=pltpu.CompilerParams(
            dimension_semantics=("arbitrary",)),
    )(tau3, tw3, beta, topic_embeddings, word_embeddings)
    return out[0, 0]
